# baseline (speedup 1.0000x reference)
import jax
import jax.numpy as jnp
from jax import lax
from jax.experimental import pallas as pl
from jax.experimental.pallas import tpu as pltpu
from jax.experimental.pallas import tpu_sc as plsc

_EMBED_DIM = 32
_BATCH = 16384

_info = plsc.get_sparse_core_info()
_NC, _NS, _NL = _info.num_cores, _info.num_subcores, _info.num_lanes
_NW = _NC * _NS
_BPW = _BATCH // _NW
_CHUNK = 128
_NCHUNK = _BPW // _CHUNK
_NGROUP = _BPW // _NL


def _emb_kernel(tidx_hbm, sidx_hbm, ttab_hbm, stab_hbm, out_hbm,
                tidx_v, sidx_v, tdst_v, sdst_v, q32_v, t128_v, trows_v,
                srows_v, sem_g, sem_sc):
    wid = lax.axis_index("s") * _NC + lax.axis_index("c")
    base = wid * _BPW

    for j in range(_NCHUNK):
        pltpu.sync_copy(tidx_hbm.at[pl.ds(base + j * _CHUNK, _CHUNK)],
                        tidx_v.at[j])
        pltpu.sync_copy(sidx_hbm.at[pl.ds(base + j * _CHUNK, _CHUNK)],
                        sidx_v.at[j])

    lane = lax.iota(jnp.int32, _NL)
    for j in range(_NCHUNK):
        for t in range(_CHUNK // _NL):
            sl = pl.ds(t * _NL, _NL)
            v = tidx_v[j, sl]
            q32_v[j, sl] = (v & 3) * _EMBED_DIM
            tidx_v[j, sl] = lax.shift_right_logical(v, 2)
            off = 2 * (base + j * _CHUNK + t * _NL) + 2 * lane
            tdst_v[j, sl] = off
            sdst_v[j, sl] = off + 1

    copies = []
    for j in range(_NCHUNK):
        copies.append(pltpu.async_copy(
            ttab_hbm.at[tidx_v.at[j]],
            t128_v.at[pl.ds(j * _CHUNK, _CHUNK)], sem_g))
        copies.append(pltpu.async_copy(
            stab_hbm.at[sidx_v.at[j]],
            srows_v.at[pl.ds(j * _CHUNK, _CHUNK)], sem_g))
    for c in copies:
        c.wait()

    @plsc.parallel_loop(0, _NGROUP, step=1, unroll=2)
    def _extract(g):
        i_vec = g * _NL + lane
        j = g // (_CHUNK // _NL)
        t = g % (_CHUNK // _NL)
        q32 = q32_v[j, pl.ds(t * _NL, _NL)]
        for d in range(_EMBED_DIM):
            vals = plsc.load_gather(t128_v, [i_vec, q32 + d])
            plsc.store_scatter(trows_v, [i_vec, lane * 0 + d], vals)

    copies = []
    for j in range(_NCHUNK):
        copies.append(pltpu.async_copy(
            trows_v.at[pl.ds(j * _CHUNK, _CHUNK)],
            out_hbm.at[tdst_v.at[j]], sem_sc))
        copies.append(pltpu.async_copy(
            srows_v.at[pl.ds(j * _CHUNK, _CHUNK)],
            out_hbm.at[sdst_v.at[j]], sem_sc))
    for c in copies:
        c.wait()


@jax.jit
def _run(tweet, sentiment, tweet_table, sentiment_table):
    mesh = plsc.VectorSubcoreMesh(core_axis_name="c", subcore_axis_name="s")
    out = pl.kernel(
        _emb_kernel,
        out_type=jax.ShapeDtypeStruct((2 * _BATCH, _EMBED_DIM), jnp.float32),
        mesh=mesh,
        compiler_params=pltpu.CompilerParams(use_tc_tiling_on_sc=False,
                                             needs_layout_passes=False),
        scratch_types=[
            pltpu.VMEM((_NCHUNK, _CHUNK), jnp.int32),
            pltpu.VMEM((_NCHUNK, _CHUNK), jnp.int32),
            pltpu.VMEM((_NCHUNK, _CHUNK), jnp.int32),
            pltpu.VMEM((_NCHUNK, _CHUNK), jnp.int32),
            pltpu.VMEM((_NCHUNK, _CHUNK), jnp.int32),
            pltpu.VMEM((_BPW, _CHUNK), jnp.float32),
            pltpu.VMEM((_BPW, _EMBED_DIM), jnp.float32),
            pltpu.VMEM((_BPW, _EMBED_DIM), jnp.float32),
            pltpu.SemaphoreType.DMA,
            pltpu.SemaphoreType.DMA,
        ],
    )(tweet, sentiment,
      tweet_table.reshape(250000, 128), sentiment_table)
    return out.reshape(_BATCH, 2 * _EMBED_DIM)


def kernel(tweet, sentiment, tweet_table, sentiment_table):
    return _run(tweet, sentiment, tweet_table, sentiment_table)


# speedup vs baseline: 1.0381x; 1.0381x over previous
import jax
import jax.numpy as jnp
from jax import lax
from jax.experimental import pallas as pl
from jax.experimental.pallas import tpu as pltpu
from jax.experimental.pallas import tpu_sc as plsc

_EMBED_DIM = 32
_BATCH = 16384

_info = plsc.get_sparse_core_info()
_NC, _NS, _NL = _info.num_cores, _info.num_subcores, _info.num_lanes
_NW = _NC * _NS
_BPW = _BATCH // _NW
_CHUNK = 128
_NCHUNK = _BPW // _CHUNK
_NGROUP = _BPW // _NL


def _emb_kernel(tidx_hbm, sidx_hbm, ttab_hbm, stab_hbm, out_hbm,
                tidx_v, sidx_v, tq32_v, t128_v, s4_v, out_v, sem_g):
    wid = lax.axis_index("s") * _NC + lax.axis_index("c")
    base = wid * _BPW

    pltpu.sync_copy(stab_hbm, s4_v)
    for j in range(_NCHUNK):
        pltpu.sync_copy(tidx_hbm.at[pl.ds(base + j * _CHUNK, _CHUNK)],
                        tidx_v.at[j])
        pltpu.sync_copy(sidx_hbm.at[pl.ds(base + j * _CHUNK, _CHUNK)],
                        sidx_v.at[j])

    for j in range(_NCHUNK):
        for t in range(_CHUNK // _NL):
            sl = pl.ds(t * _NL, _NL)
            v = tidx_v[j, sl]
            tq32_v[j, sl] = (v & 3) * _EMBED_DIM
            tidx_v[j, sl] = lax.shift_right_logical(v, 2)

    copies = []
    for j in range(_NCHUNK):
        copies.append(pltpu.async_copy(
            ttab_hbm.at[tidx_v.at[j]],
            t128_v.at[pl.ds(j * _CHUNK, _CHUNK)], sem_g))
    for c in copies:
        c.wait()

    lane = lax.iota(jnp.int32, _NL)
    @plsc.parallel_loop(0, _NGROUP, step=1, unroll=2)
    def _assemble(g):
        i_vec = g * _NL + lane
        j = g // (_CHUNK // _NL)
        t = g % (_CHUNK // _NL)
        sl = pl.ds(t * _NL, _NL)
        tq32 = tq32_v[j, sl]
        s = sidx_v[j, sl]
        srow = lax.shift_right_logical(s, 2)
        sq32 = (s & 3) * _EMBED_DIM
        orow = lax.shift_right_logical(i_vec, 1)
        obase = (i_vec & 1) * (2 * _EMBED_DIM)
        for d in range(_EMBED_DIM):
            tv = plsc.load_gather(t128_v, [i_vec, tq32 + d])
            plsc.store_scatter(out_v, [orow, obase + d], tv)
            sv = plsc.load_gather(s4_v, [srow, sq32 + d])
            plsc.store_scatter(out_v, [orow, obase + _EMBED_DIM + d], sv)

    pltpu.sync_copy(out_v, out_hbm.at[pl.ds(wid * (_BPW // 2), _BPW // 2)])


@jax.jit
def _run(tweet, sentiment, tweet_table, sentiment_table):
    mesh = plsc.VectorSubcoreMesh(core_axis_name="c", subcore_axis_name="s")
    out = pl.kernel(
        _emb_kernel,
        out_type=jax.ShapeDtypeStruct((_BATCH // 2, 2 * _CHUNK // 2),
                                      jnp.float32),
        mesh=mesh,
        compiler_params=pltpu.CompilerParams(use_tc_tiling_on_sc=True,
                                             needs_layout_passes=False),
        scratch_types=[
            pltpu.VMEM((_NCHUNK, _CHUNK), jnp.int32),
            pltpu.VMEM((_NCHUNK, _CHUNK), jnp.int32),
            pltpu.VMEM((_NCHUNK, _CHUNK), jnp.int32),
            pltpu.VMEM((_BPW, _CHUNK), jnp.float32),
            pltpu.VMEM((4, _CHUNK), jnp.float32),
            pltpu.VMEM((_BPW // 2, _CHUNK), jnp.float32),
            pltpu.SemaphoreType.DMA,
        ],
    )(tweet, sentiment,
      tweet_table.reshape(250000, 128), sentiment_table.reshape(4, 128))
    return out.reshape(_BATCH, 2 * _EMBED_DIM)


def kernel(tweet, sentiment, tweet_table, sentiment_table):
    return _run(tweet, sentiment, tweet_table, sentiment_table)


# speedup vs baseline: 1.0383x; 1.0001x over previous
import jax
import jax.numpy as jnp
from jax import lax
from jax.experimental import pallas as pl
from jax.experimental.pallas import tpu as pltpu
from jax.experimental.pallas import tpu_sc as plsc

_EMBED_DIM = 32
_BATCH = 16384

_info = plsc.get_sparse_core_info()
_NC, _NS, _NL = _info.num_cores, _info.num_subcores, _info.num_lanes
_NW = _NC * _NS
_BPW = _BATCH // _NW
_CHUNK = 128
_NCHUNK = _BPW // _CHUNK
_NGROUP = _BPW // _NL


def _emb_kernel(tidx_hbm, sidx_hbm, ttab_hbm, stab_hbm, out_hbm,
                tidx_v, sidx_v, tq32_v, t128_v, s4_v, out_v, sem_g):
    wid = lax.axis_index("s") * _NC + lax.axis_index("c")
    base = wid * _BPW

    pltpu.sync_copy(stab_hbm, s4_v)
    for j in range(_NCHUNK):
        pltpu.sync_copy(tidx_hbm.at[pl.ds(base + j * _CHUNK, _CHUNK)],
                        tidx_v.at[j])
        pltpu.sync_copy(sidx_hbm.at[pl.ds(base + j * _CHUNK, _CHUNK)],
                        sidx_v.at[j])

    for j in range(_NCHUNK):
        for t in range(_CHUNK // _NL):
            sl = pl.ds(t * _NL, _NL)
            v = tidx_v[j, sl]
            tq32_v[j, sl] = (v & 3) * _EMBED_DIM
            tidx_v[j, sl] = lax.shift_right_logical(v, 2)

    copies = []
    for j in range(_NCHUNK):
        copies.append(pltpu.async_copy(
            ttab_hbm.at[tidx_v.at[j]],
            t128_v.at[pl.ds(j * _CHUNK, _CHUNK)], sem_g))
    for c in copies:
        c.wait()

    lane = lax.iota(jnp.int32, _NL)
    @plsc.parallel_loop(0, _NGROUP, step=1, unroll=2)
    def _assemble(g):
        i_vec = g * _NL + lane
        j = g // (_CHUNK // _NL)
        t = g % (_CHUNK // _NL)
        sl = pl.ds(t * _NL, _NL)
        tq32 = tq32_v[j, sl]
        s = sidx_v[j, sl]
        srow = lax.shift_right_logical(s, 2)
        sq32 = (s & 3) * _EMBED_DIM
        orow = lax.shift_right_logical(i_vec, 1)
        obase = (i_vec & 1) * (2 * _EMBED_DIM)
        for d in range(_EMBED_DIM):
            tv = plsc.load_gather(t128_v, [i_vec, tq32 + d])
            plsc.store_scatter(out_v, [orow, obase + d], tv)
            sv = plsc.load_gather(s4_v, [srow, sq32 + d])
            plsc.store_scatter(out_v, [orow, obase + _EMBED_DIM + d], sv)

    pltpu.sync_copy(out_v, out_hbm.at[pl.ds(wid * (_BPW // 2), _BPW // 2)])


@jax.jit
def _run(tweet, sentiment, tweet_table, sentiment_table):
    mesh = plsc.VectorSubcoreMesh(core_axis_name="c", subcore_axis_name="s")
    out = pl.kernel(
        _emb_kernel,
        out_type=jax.ShapeDtypeStruct((_BATCH // 2, 2 * _CHUNK // 2),
                                      jnp.float32),
        mesh=mesh,
        compiler_params=pltpu.CompilerParams(use_tc_tiling_on_sc=True,
                                             needs_layout_passes=False,
                                             skip_device_barrier=True),
        scratch_types=[
            pltpu.VMEM((_NCHUNK, _CHUNK), jnp.int32),
            pltpu.VMEM((_NCHUNK, _CHUNK), jnp.int32),
            pltpu.VMEM((_NCHUNK, _CHUNK), jnp.int32),
            pltpu.VMEM((_BPW, _CHUNK), jnp.float32),
            pltpu.VMEM((4, _CHUNK), jnp.float32),
            pltpu.VMEM((_BPW // 2, _CHUNK), jnp.float32),
            pltpu.SemaphoreType.DMA,
        ],
    )(tweet, sentiment,
      tweet_table.reshape(250000, 128), sentiment_table.reshape(4, 128))
    return out.reshape(_BATCH, 2 * _EMBED_DIM)


def kernel(tweet, sentiment, tweet_table, sentiment_table):
    return _run(tweet, sentiment, tweet_table, sentiment_table)
